# Initial kernel scaffold; baseline (speedup 1.0000x reference)
import jax
import jax.numpy as jnp
from jax.experimental import pallas as pl


def kernel(x, edge_index, batch, eigen_values, W_l0, b_l0, W_r0, W_l1, b_l1, W_r1, fc1_w, fc1_b, fc2_w, fc2_b):
    raise NotImplementedError("write your pallas kernel here")


# speedup vs baseline: 3.1306x; 3.1306x over previous
import functools

import jax
import jax.numpy as jnp
from jax import lax
from jax.experimental import pallas as pl
from jax.experimental.pallas import tpu as pltpu
from jax.experimental.pallas import tpu_sc as plsc

_N = 10000
_E = 320000
_G = 64
_DF = 128
_DI = 32
_OH = 224
_FB = _DI + 16
_FH = _OH // 2

_NC = 2
_NS = 16
_NW = _NC * _NS
_CH = 128
_EPW = 10112
_EPAD = _NW * _EPW
_NPAD = 10112
_RPW = _NPAD // _NS

_RT = 2000


def _make_seg_sum(F):
  mesh = plsc.VectorSubcoreMesh(
      core_axis_name="c", subcore_axis_name="s",
      num_cores=_NC, num_subcores=_NS)

  @functools.partial(
      pl.kernel,
      out_type=(jax.ShapeDtypeStruct((_NPAD, F), jnp.float32),
                jax.ShapeDtypeStruct((_NPAD, F), jnp.float32)),
      mesh=mesh,
      compiler_params=pltpu.CompilerParams(use_tc_tiling_on_sc=False),
      scratch_types=[
          pltpu.VMEM((_CH,), jnp.int32),
          pltpu.VMEM((_CH,), jnp.int32),
          pltpu.VMEM((_CH, F), jnp.float32),
          pltpu.VMEM_SHARED((_NPAD, F), jnp.float32),
          pltpu.SemaphoreType.DMA,
      ])
  def seg_sum(src_hbm, dst_hbm, table_hbm, zeros_hbm, out0, out1,
              src_v, dst_v, rows_v, acc_sh, sem):
    cid = lax.axis_index("c")
    sid = lax.axis_index("s")
    wid = cid * _NS + sid
    r0 = sid * _RPW
    pltpu.sync_copy(zeros_hbm.at[pl.ds(r0, _RPW)], acc_sh.at[pl.ds(r0, _RPW)])
    plsc.subcore_barrier()

    e0 = wid * _EPW

    def body(i, carry):
      base = e0 + i * _CH
      pltpu.sync_copy(src_hbm.at[pl.ds(base, _CH)], src_v)
      pltpu.sync_copy(dst_hbm.at[pl.ds(base, _CH)], dst_v)
      pltpu.async_copy(table_hbm.at[src_v], rows_v, sem).wait()
      pltpu.sync_copy(rows_v, acc_sh.at[dst_v], add=True)
      return carry

    lax.fori_loop(0, _EPW // _CH, body, 0)
    plsc.subcore_barrier()

    @pl.when(cid == 0)
    def _():
      pltpu.sync_copy(acc_sh.at[pl.ds(r0, _RPW)], out0.at[pl.ds(r0, _RPW)])

    @pl.when(cid == 1)
    def _():
      pltpu.sync_copy(acc_sh.at[pl.ds(r0, _RPW)], out1.at[pl.ds(r0, _RPW)])

  return seg_sum


_seg_sum_x = _make_seg_sum(_DF)
_seg_sum_b = _make_seg_sum(_FB)
_seg_sum_h = _make_seg_sum(_FH)


def _prep_body(bc_ref, ev_ref, out_ref):
  onehot = (bc_ref[...]
            == lax.broadcasted_iota(jnp.int32, (1, _G), 1)).astype(jnp.float32)
  rep = jnp.dot(onehot, ev_ref[...], preferred_element_type=jnp.float32)
  onescol = (lax.broadcasted_iota(jnp.int32, (_RT, 16), 1) == 0
             ).astype(jnp.float32)
  out_ref[...] = jnp.concatenate([rep, onescol], axis=1)


def _l0_body(p0x_ref, p1x_ref, p0b_ref, p1b_ref, tb_ref, x_ref,
             wlx_ref, wlr_ref, wrx_ref, wrr_ref, b_ref,
             h0a_ref, h0b_ref, aux_ref):
  sx = p0x_ref[...] + p1x_ref[...]
  sb = p0b_ref[...] + p1b_ref[...]
  dinv = 1.0 / jnp.maximum(sb[:, _DI:_DI + 1], 1.0)
  mrep = sb[:, 0:_DI] * dinv
  h0 = (jnp.dot(sx * dinv, wlx_ref[...], preferred_element_type=jnp.float32)
        + jnp.dot(mrep, wlr_ref[...], preferred_element_type=jnp.float32)
        + jnp.dot(x_ref[...], wrx_ref[...], preferred_element_type=jnp.float32)
        + jnp.dot(tb_ref[:, 0:_DI], wrr_ref[...],
                  preferred_element_type=jnp.float32)
        + b_ref[...])
  h0a_ref[...] = h0[:, :_FH]
  h0b_ref[...] = h0[:, _FH:]
  aux_ref[...] = jnp.concatenate(
      [mrep, jnp.broadcast_to(dinv, (_RT, 32))], axis=1)


def _l1_body(q0a_ref, q1a_ref, q0b_ref, q1b_ref, aux_ref, h0a_ref, h0b_ref,
             tb_ref, wlh_ref, wlr_ref, wrh_ref, wrr_ref, b_ref, out_ref):
  dinv = aux_ref[:, 32:33]
  mrep = aux_ref[:, 0:32]
  meanh = jnp.concatenate([(q0a_ref[...] + q1a_ref[...]) * dinv,
                           (q0b_ref[...] + q1b_ref[...]) * dinv], axis=1)
  h0 = jnp.concatenate([h0a_ref[...], h0b_ref[...]], axis=1)
  rep = tb_ref[:, 0:_DI]
  out_ref[...] = (
      jnp.dot(meanh, wlh_ref[...], preferred_element_type=jnp.float32)
      + jnp.dot(mrep, wlr_ref[...], preferred_element_type=jnp.float32)
      + jnp.dot(h0, wrh_ref[...], preferred_element_type=jnp.float32)
      + jnp.dot(rep, wrr_ref[...], preferred_element_type=jnp.float32)
      + b_ref[...])


def _pool_body(bc_ref, h0a_ref, h0b_ref, h1_ref,
               fc1w_ref, fc1b_ref, fc2w_ref, fc2b_ref, out_ref, pool_ref):
  i = pl.program_id(0)

  @pl.when(i == 0)
  def _():
    pool_ref[...] = jnp.full((_G, 2 * _OH), -jnp.inf, jnp.float32)

  hcat = jnp.concatenate([h0a_ref[...], h0b_ref[...], h1_ref[...]], axis=1)
  bc = bc_ref[...]

  def gbody(g, carry):
    red = jnp.max(jnp.where(bc == g, hcat, -jnp.inf), axis=0, keepdims=True)
    pool_ref[pl.ds(g, 1), :] = jnp.maximum(pool_ref[pl.ds(g, 1), :], red)
    return carry

  lax.fori_loop(0, _G, gbody, 0)

  @pl.when(i == pl.num_programs(0) - 1)
  def _():
    z = jnp.maximum(
        jnp.dot(pool_ref[...], fc1w_ref[...],
                preferred_element_type=jnp.float32) + fc1b_ref[...], 0.0)
    out_ref[...] = (jnp.dot(z, fc2w_ref[...],
                            preferred_element_type=jnp.float32)
                    + fc2b_ref[...])


def kernel(x, edge_index, batch, eigen_values,
           W_l0, b_l0, W_r0, W_l1, b_l1, W_r1,
           fc1_w, fc1_b, fc2_w, fc2_b):
  ng = _N // _RT
  srcp = jnp.concatenate(
      [edge_index[0], jnp.zeros((_EPAD - _E,), jnp.int32)])
  dstp = jnp.concatenate(
      [edge_index[1], jnp.full((_EPAD - _E,), _N, jnp.int32)])
  bc = batch.reshape(_N, 1)
  z_x = jnp.zeros((_NPAD, _DF), jnp.float32)
  z_b = jnp.zeros((_NPAD, _FB), jnp.float32)
  z_h = jnp.zeros((_NPAD, _FH), jnp.float32)
  b_l0r = b_l0.reshape(1, _OH)
  b_l1r = b_l1.reshape(1, _OH)
  fc1_br = fc1_b.reshape(1, -1)
  fc2_br = fc2_b.reshape(1, -1)
  wl0x, wl0r = W_l0[:_DF], W_l0[_DF:]
  wr0x, wr0r = W_r0[:_DF], W_r0[_DF:]
  wl1h, wl1r = W_l1[:_OH], W_l1[_OH:]
  wr1h, wr1r = W_r1[:_OH], W_r1[_OH:]

  tb = pl.pallas_call(
      _prep_body,
      grid=(ng,),
      in_specs=[
          pl.BlockSpec((_RT, 1), lambda i: (i, 0)),
          pl.BlockSpec((_G, _DI), lambda i: (0, 0)),
      ],
      out_specs=pl.BlockSpec((_RT, _FB), lambda i: (i, 0)),
      out_shape=jax.ShapeDtypeStruct((_N, _FB), jnp.float32),
  )(bc, eigen_values)

  p0x, p1x = _seg_sum_x(srcp, dstp, x, z_x)
  p0b, p1b = _seg_sum_b(srcp, dstp, tb, z_b)

  wspec = lambda r, c: pl.BlockSpec((r, c), lambda i: (0, 0))
  rspec = lambda c: pl.BlockSpec((_RT, c), lambda i: (i, 0))
  h0a, h0b, aux = pl.pallas_call(
      _l0_body,
      grid=(ng,),
      in_specs=[
          rspec(_DF), rspec(_DF), rspec(_FB), rspec(_FB), rspec(_FB),
          rspec(_DF),
          wspec(_DF, _OH), wspec(_DI, _OH),
          wspec(_DF, _OH), wspec(_DI, _OH), wspec(1, _OH),
      ],
      out_specs=[rspec(_FH), rspec(_FH), rspec(64)],
      out_shape=[
          jax.ShapeDtypeStruct((_N, _FH), jnp.float32),
          jax.ShapeDtypeStruct((_N, _FH), jnp.float32),
          jax.ShapeDtypeStruct((_N, 64), jnp.float32),
      ],
  )(p0x, p1x, p0b, p1b, tb, x, wl0x, wl0r, wr0x, wr0r, b_l0r)

  q0a, q1a = _seg_sum_h(srcp, dstp, h0a, z_h)
  q0b, q1b = _seg_sum_h(srcp, dstp, h0b, z_h)

  h1 = pl.pallas_call(
      _l1_body,
      grid=(ng,),
      in_specs=[
          rspec(_FH), rspec(_FH), rspec(_FH), rspec(_FH),
          rspec(64), rspec(_FH), rspec(_FH), rspec(_FB),
          wspec(_OH, _OH), wspec(_DI, _OH),
          wspec(_OH, _OH), wspec(_DI, _OH), wspec(1, _OH),
      ],
      out_specs=rspec(_OH),
      out_shape=jax.ShapeDtypeStruct((_N, _OH), jnp.float32),
  )(q0a, q1a, q0b, q1b, aux, h0a, h0b, tb,
    wl1h, wl1r, wr1h, wr1r, b_l1r)

  out = pl.pallas_call(
      _pool_body,
      grid=(ng,),
      in_specs=[
          pl.BlockSpec((_RT, 1), lambda i: (i, 0)),
          rspec(_FH), rspec(_FH), rspec(_OH),
          wspec(2 * _OH, 256), wspec(1, 256),
          wspec(256, 10), wspec(1, 10),
      ],
      out_specs=pl.BlockSpec((_G, 10), lambda i: (0, 0)),
      out_shape=jax.ShapeDtypeStruct((_G, 10), jnp.float32),
      scratch_shapes=[pltpu.VMEM((_G, 2 * _OH), jnp.float32)],
  )(bc, h0a, h0b, h1, fc1_w, fc1_br, fc2_w, fc2_br)
  return out


# speedup vs baseline: 4.0984x; 1.3091x over previous
import functools

import jax
import jax.numpy as jnp
from jax import lax
from jax.experimental import pallas as pl
from jax.experimental.pallas import tpu as pltpu
from jax.experimental.pallas import tpu_sc as plsc

_N = 10000
_E = 320000
_G = 64
_DF = 128
_DI = 32
_OH = 224
_FB = _DI + 16
_FH = _OH // 2

_NC = 2
_NS = 16
_NW = _NC * _NS
_CH = 128
_EPW = 10112
_EPAD = _NW * _EPW
_NPAD = 10112
_RPW = _NPAD // _NS

_RT = 2000


def _make_seg_sum(F):
  mesh = plsc.VectorSubcoreMesh(
      core_axis_name="c", subcore_axis_name="s",
      num_cores=_NC, num_subcores=_NS)

  @functools.partial(
      pl.kernel,
      out_type=(jax.ShapeDtypeStruct((_NPAD, F), jnp.float32),
                jax.ShapeDtypeStruct((_NPAD, F), jnp.float32)),
      mesh=mesh,
      compiler_params=pltpu.CompilerParams(use_tc_tiling_on_sc=False),
      scratch_types=[
          pltpu.VMEM((_CH,), jnp.int32),
          pltpu.VMEM((_CH,), jnp.int32),
          pltpu.VMEM((_CH,), jnp.int32),
          pltpu.VMEM((_CH,), jnp.int32),
          pltpu.VMEM((_CH, F), jnp.float32),
          pltpu.VMEM((_CH, F), jnp.float32),
          pltpu.SemaphoreType.DMA,
          pltpu.SemaphoreType.DMA,
          pltpu.SemaphoreType.DMA,
          pltpu.SemaphoreType.DMA,
          pltpu.VMEM_SHARED((_NPAD, F), jnp.float32),
      ])
  def seg_sum(src_hbm, dst_hbm, table_hbm, zeros_hbm, out0, out1,
              src_a, dst_a, src_b, dst_b, rows_a, rows_b,
              gsem_a, gsem_b, ssem_a, ssem_b, acc_sh):
    cid = lax.axis_index("c")
    sid = lax.axis_index("s")
    wid = cid * _NS + sid
    r0 = sid * _RPW
    pltpu.sync_copy(zeros_hbm.at[pl.ds(r0, _RPW)], acc_sh.at[pl.ds(r0, _RPW)])
    plsc.subcore_barrier()

    e0 = wid * _EPW

    def half(base, src_v, dst_v, rows_v, gsem, ssem,
             o_dst_v, o_rows_v, o_gsem, o_ssem, first_buf, first_prev):
      @pl.when(jnp.logical_not(first_buf))
      def _():
        pltpu.make_async_copy(rows_v, acc_sh.at[dst_v], ssem).wait()
      pltpu.sync_copy(src_hbm.at[pl.ds(base, _CH)], src_v)
      pltpu.sync_copy(dst_hbm.at[pl.ds(base, _CH)], dst_v)
      pltpu.async_copy(table_hbm.at[src_v], rows_v, gsem)

      @pl.when(jnp.logical_not(first_prev))
      def _():
        pltpu.make_async_copy(table_hbm.at[0:_CH], o_rows_v, o_gsem).wait()
        pltpu.async_copy(o_rows_v, acc_sh.at[o_dst_v], o_ssem, add=True)

    def body(j, carry):
      base = e0 + j * (2 * _CH)
      first = j == 0
      half(base, src_a, dst_a, rows_a, gsem_a, ssem_a,
           dst_b, rows_b, gsem_b, ssem_b, first, first)
      half(base + _CH, src_b, dst_b, rows_b, gsem_b, ssem_b,
           dst_a, rows_a, gsem_a, ssem_a, first, jnp.bool_(False))
      return carry

    npair = (_EPW // _CH - 1) // 2
    lax.fori_loop(0, npair, body, 0)
    base = e0 + (_EPW - _CH)
    pltpu.make_async_copy(rows_a, acc_sh.at[dst_a], ssem_a).wait()
    pltpu.sync_copy(src_hbm.at[pl.ds(base, _CH)], src_a)
    pltpu.sync_copy(dst_hbm.at[pl.ds(base, _CH)], dst_a)
    pltpu.async_copy(table_hbm.at[src_a], rows_a, gsem_a)
    pltpu.make_async_copy(table_hbm.at[0:_CH], rows_b, gsem_b).wait()
    pltpu.async_copy(rows_b, acc_sh.at[dst_b], ssem_b, add=True)
    pltpu.make_async_copy(table_hbm.at[0:_CH], rows_a, gsem_a).wait()
    pltpu.async_copy(rows_a, acc_sh.at[dst_a], ssem_a, add=True)
    pltpu.make_async_copy(rows_a, acc_sh.at[dst_a], ssem_a).wait()
    pltpu.make_async_copy(rows_b, acc_sh.at[dst_b], ssem_b).wait()
    plsc.subcore_barrier()

    @pl.when(cid == 0)
    def _():
      pltpu.sync_copy(acc_sh.at[pl.ds(r0, _RPW)], out0.at[pl.ds(r0, _RPW)])

    @pl.when(cid == 1)
    def _():
      pltpu.sync_copy(acc_sh.at[pl.ds(r0, _RPW)], out1.at[pl.ds(r0, _RPW)])

  return seg_sum


_seg_sum_x = _make_seg_sum(_DF)
_seg_sum_b = _make_seg_sum(_FB)
_seg_sum_h = _make_seg_sum(_FH)


def _prep_body(bc_ref, ev_ref, out_ref):
  onehot = (bc_ref[...]
            == lax.broadcasted_iota(jnp.int32, (1, _G), 1)).astype(jnp.float32)
  rep = jnp.dot(onehot, ev_ref[...], preferred_element_type=jnp.float32)
  onescol = (lax.broadcasted_iota(jnp.int32, (_RT, 16), 1) == 0
             ).astype(jnp.float32)
  out_ref[...] = jnp.concatenate([rep, onescol], axis=1)


def _l0_body(p0x_ref, p1x_ref, p0b_ref, p1b_ref, tb_ref, x_ref,
             wlx_ref, wlr_ref, wrx_ref, wrr_ref, b_ref,
             h0a_ref, h0b_ref, aux_ref):
  sx = p0x_ref[...] + p1x_ref[...]
  sb = p0b_ref[...] + p1b_ref[...]
  dinv = 1.0 / jnp.maximum(sb[:, _DI:_DI + 1], 1.0)
  mrep = sb[:, 0:_DI] * dinv
  h0 = (jnp.dot(sx * dinv, wlx_ref[...], preferred_element_type=jnp.float32)
        + jnp.dot(mrep, wlr_ref[...], preferred_element_type=jnp.float32)
        + jnp.dot(x_ref[...], wrx_ref[...], preferred_element_type=jnp.float32)
        + jnp.dot(tb_ref[:, 0:_DI], wrr_ref[...],
                  preferred_element_type=jnp.float32)
        + b_ref[...])
  h0a_ref[...] = h0[:, :_FH]
  h0b_ref[...] = h0[:, _FH:]
  aux_ref[...] = jnp.concatenate(
      [mrep, jnp.broadcast_to(dinv, (_RT, 32))], axis=1)


def _l1_body(q0a_ref, q1a_ref, q0b_ref, q1b_ref, aux_ref, h0a_ref, h0b_ref,
             tb_ref, wlh_ref, wlr_ref, wrh_ref, wrr_ref, b_ref, out_ref):
  dinv = aux_ref[:, 32:33]
  mrep = aux_ref[:, 0:32]
  meanh = jnp.concatenate([(q0a_ref[...] + q1a_ref[...]) * dinv,
                           (q0b_ref[...] + q1b_ref[...]) * dinv], axis=1)
  h0 = jnp.concatenate([h0a_ref[...], h0b_ref[...]], axis=1)
  rep = tb_ref[:, 0:_DI]
  out_ref[...] = (
      jnp.dot(meanh, wlh_ref[...], preferred_element_type=jnp.float32)
      + jnp.dot(mrep, wlr_ref[...], preferred_element_type=jnp.float32)
      + jnp.dot(h0, wrh_ref[...], preferred_element_type=jnp.float32)
      + jnp.dot(rep, wrr_ref[...], preferred_element_type=jnp.float32)
      + b_ref[...])


def _pool_body(bc_ref, h0a_ref, h0b_ref, h1_ref,
               fc1w_ref, fc1b_ref, fc2w_ref, fc2b_ref, out_ref, pool_ref):
  i = pl.program_id(0)

  @pl.when(i == 0)
  def _():
    pool_ref[...] = jnp.full((_G, 2 * _OH), -jnp.inf, jnp.float32)

  hcat = jnp.concatenate([h0a_ref[...], h0b_ref[...], h1_ref[...]], axis=1)
  bc = bc_ref[...]

  def gbody(g, carry):
    red = jnp.max(jnp.where(bc == g, hcat, -jnp.inf), axis=0, keepdims=True)
    pool_ref[pl.ds(g, 1), :] = jnp.maximum(pool_ref[pl.ds(g, 1), :], red)
    return carry

  lax.fori_loop(0, _G, gbody, 0)

  @pl.when(i == pl.num_programs(0) - 1)
  def _():
    z = jnp.maximum(
        jnp.dot(pool_ref[...], fc1w_ref[...],
                preferred_element_type=jnp.float32) + fc1b_ref[...], 0.0)
    out_ref[...] = (jnp.dot(z, fc2w_ref[...],
                            preferred_element_type=jnp.float32)
                    + fc2b_ref[...])


def kernel(x, edge_index, batch, eigen_values,
           W_l0, b_l0, W_r0, W_l1, b_l1, W_r1,
           fc1_w, fc1_b, fc2_w, fc2_b):
  ng = _N // _RT
  srcp = jnp.concatenate(
      [edge_index[0], jnp.zeros((_EPAD - _E,), jnp.int32)])
  dstp = jnp.concatenate(
      [edge_index[1], jnp.full((_EPAD - _E,), _N, jnp.int32)])
  bc = batch.reshape(_N, 1)
  z_x = jnp.zeros((_NPAD, _DF), jnp.float32)
  z_b = jnp.zeros((_NPAD, _FB), jnp.float32)
  z_h = jnp.zeros((_NPAD, _FH), jnp.float32)
  b_l0r = b_l0.reshape(1, _OH)
  b_l1r = b_l1.reshape(1, _OH)
  fc1_br = fc1_b.reshape(1, -1)
  fc2_br = fc2_b.reshape(1, -1)
  wl0x, wl0r = W_l0[:_DF], W_l0[_DF:]
  wr0x, wr0r = W_r0[:_DF], W_r0[_DF:]
  wl1h, wl1r = W_l1[:_OH], W_l1[_OH:]
  wr1h, wr1r = W_r1[:_OH], W_r1[_OH:]

  tb = pl.pallas_call(
      _prep_body,
      grid=(ng,),
      in_specs=[
          pl.BlockSpec((_RT, 1), lambda i: (i, 0)),
          pl.BlockSpec((_G, _DI), lambda i: (0, 0)),
      ],
      out_specs=pl.BlockSpec((_RT, _FB), lambda i: (i, 0)),
      out_shape=jax.ShapeDtypeStruct((_N, _FB), jnp.float32),
  )(bc, eigen_values)

  p0x, p1x = _seg_sum_x(srcp, dstp, x, z_x)
  p0b, p1b = _seg_sum_b(srcp, dstp, tb, z_b)

  wspec = lambda r, c: pl.BlockSpec((r, c), lambda i: (0, 0))
  rspec = lambda c: pl.BlockSpec((_RT, c), lambda i: (i, 0))
  h0a, h0b, aux = pl.pallas_call(
      _l0_body,
      grid=(ng,),
      in_specs=[
          rspec(_DF), rspec(_DF), rspec(_FB), rspec(_FB), rspec(_FB),
          rspec(_DF),
          wspec(_DF, _OH), wspec(_DI, _OH),
          wspec(_DF, _OH), wspec(_DI, _OH), wspec(1, _OH),
      ],
      out_specs=[rspec(_FH), rspec(_FH), rspec(64)],
      out_shape=[
          jax.ShapeDtypeStruct((_N, _FH), jnp.float32),
          jax.ShapeDtypeStruct((_N, _FH), jnp.float32),
          jax.ShapeDtypeStruct((_N, 64), jnp.float32),
      ],
  )(p0x, p1x, p0b, p1b, tb, x, wl0x, wl0r, wr0x, wr0r, b_l0r)

  q0a, q1a = _seg_sum_h(srcp, dstp, h0a, z_h)
  q0b, q1b = _seg_sum_h(srcp, dstp, h0b, z_h)

  h1 = pl.pallas_call(
      _l1_body,
      grid=(ng,),
      in_specs=[
          rspec(_FH), rspec(_FH), rspec(_FH), rspec(_FH),
          rspec(64), rspec(_FH), rspec(_FH), rspec(_FB),
          wspec(_OH, _OH), wspec(_DI, _OH),
          wspec(_OH, _OH), wspec(_DI, _OH), wspec(1, _OH),
      ],
      out_specs=rspec(_OH),
      out_shape=jax.ShapeDtypeStruct((_N, _OH), jnp.float32),
  )(q0a, q1a, q0b, q1b, aux, h0a, h0b, tb,
    wl1h, wl1r, wr1h, wr1r, b_l1r)

  out = pl.pallas_call(
      _pool_body,
      grid=(ng,),
      in_specs=[
          pl.BlockSpec((_RT, 1), lambda i: (i, 0)),
          rspec(_FH), rspec(_FH), rspec(_OH),
          wspec(2 * _OH, 256), wspec(1, 256),
          wspec(256, 10), wspec(1, 10),
      ],
      out_specs=pl.BlockSpec((_G, 10), lambda i: (0, 0)),
      out_shape=jax.ShapeDtypeStruct((_G, 10), jnp.float32),
      scratch_shapes=[pltpu.VMEM((_G, 2 * _OH), jnp.float32)],
  )(bc, h0a, h0b, h1, fc1_w, fc1_br, fc2_w, fc2_br)
  return out


# speedup vs baseline: 4.4521x; 1.0863x over previous
import functools

import jax
import jax.numpy as jnp
from jax import lax
from jax.experimental import pallas as pl
from jax.experimental.pallas import tpu as pltpu
from jax.experimental.pallas import tpu_sc as plsc

_N = 10000
_E = 320000
_G = 64
_DF = 128
_DI = 32
_OH = 224
_FB = _DI + 16
_FH = _OH // 2

_NC = 2
_NS = 16
_NW = _NC * _NS
_CH = 128
_EPW = 10112
_EPAD = _NW * _EPW
_NCH = _EPW // _CH
_NPAD = 10112
_RPW = _NPAD // _NS

_RT = 2000


def _make_seg_sum(F):
  mesh = plsc.VectorSubcoreMesh(
      core_axis_name="c", subcore_axis_name="s",
      num_cores=_NC, num_subcores=_NS)

  @functools.partial(
      pl.kernel,
      out_type=(jax.ShapeDtypeStruct((_NPAD, F), jnp.float32),
                jax.ShapeDtypeStruct((_NPAD, F), jnp.float32)),
      mesh=mesh,
      compiler_params=pltpu.CompilerParams(use_tc_tiling_on_sc=False),
      scratch_types=[
          pltpu.VMEM((_CH,), jnp.int32),
          pltpu.VMEM((_CH,), jnp.int32),
          pltpu.VMEM((_CH,), jnp.int32),
          pltpu.VMEM((_CH,), jnp.int32),
          pltpu.VMEM((_CH,), jnp.int32),
          pltpu.VMEM((_CH,), jnp.int32),
          pltpu.VMEM((_CH, F), jnp.float32),
          pltpu.VMEM((_CH, F), jnp.float32),
          pltpu.VMEM((_CH, F), jnp.float32),
          pltpu.SemaphoreType.DMA,
          pltpu.SemaphoreType.DMA,
          pltpu.SemaphoreType.DMA,
          pltpu.SemaphoreType.DMA,
          pltpu.SemaphoreType.DMA,
          pltpu.SemaphoreType.DMA,
          pltpu.SemaphoreType.DMA,
          pltpu.SemaphoreType.DMA,
          pltpu.SemaphoreType.DMA,
          pltpu.VMEM_SHARED((_NPAD, F), jnp.float32),
      ])
  def seg_sum(src_hbm, dst_hbm, table_hbm, zeros_hbm, out0, out1,
              src_0, src_1, src_2, dst_0, dst_1, dst_2,
              rows_0, rows_1, rows_2,
              gsem_0, gsem_1, gsem_2, ssem_0, ssem_1, ssem_2,
              isem_0, isem_1, isem_2, acc_sh):
    srcv = (src_0, src_1, src_2)
    dstv = (dst_0, dst_1, dst_2)
    rows = (rows_0, rows_1, rows_2)
    gsem = (gsem_0, gsem_1, gsem_2)
    ssem = (ssem_0, ssem_1, ssem_2)
    isem = (isem_0, isem_1, isem_2)
    cid = lax.axis_index("c")
    sid = lax.axis_index("s")
    wid = cid * _NS + sid
    e0 = wid * _EPW

    def fetch(c, k):
      pltpu.async_copy(src_hbm.at[pl.ds(e0 + c * _CH, _CH)], srcv[k], isem[k])
      pltpu.async_copy(dst_hbm.at[pl.ds(e0 + c * _CH, _CH)], dstv[k], isem[k])

    def fetch_wait(k):
      pltpu.make_async_copy(src_hbm.at[0:_CH], srcv[k], isem[k]).wait()
      pltpu.make_async_copy(src_hbm.at[0:_CH], dstv[k], isem[k]).wait()

    fetch(0, 0)
    r0 = sid * _RPW
    pltpu.sync_copy(zeros_hbm.at[pl.ds(r0, _RPW)], acc_sh.at[pl.ds(r0, _RPW)])
    plsc.subcore_barrier()

    def step(c, k, kp, kn, g2, g1):
      fetch_wait(k)
      pltpu.async_copy(table_hbm.at[srcv[k]], rows[k], gsem[k])

      @pl.when(g2)
      def _():
        pltpu.make_async_copy(rows[kn], acc_sh.at[dstv[kn]], ssem[kn]).wait()
      fetch(c + 1, kn)

      @pl.when(g1)
      def _():
        pltpu.make_async_copy(table_hbm.at[0:_CH], rows[kp], gsem[kp]).wait()
        pltpu.async_copy(rows[kp], acc_sh.at[dstv[kp]], ssem[kp], add=True)

    def body(j, carry):
      nf = j > 0
      step(3 * j, 0, 2, 1, nf, nf)
      step(3 * j + 1, 1, 0, 2, nf, jnp.bool_(True))
      step(3 * j + 2, 2, 1, 0, jnp.bool_(True), jnp.bool_(True))
      return carry

    lax.fori_loop(0, (_NCH - 1) // 3, body, 0)
    fetch_wait(0)
    pltpu.async_copy(table_hbm.at[srcv[0]], rows[0], gsem[0])
    pltpu.make_async_copy(rows[1], acc_sh.at[dstv[1]], ssem[1]).wait()
    pltpu.make_async_copy(table_hbm.at[0:_CH], rows[2], gsem[2]).wait()
    pltpu.async_copy(rows[2], acc_sh.at[dstv[2]], ssem[2], add=True)
    pltpu.make_async_copy(table_hbm.at[0:_CH], rows[0], gsem[0]).wait()
    pltpu.async_copy(rows[0], acc_sh.at[dstv[0]], ssem[0], add=True)
    pltpu.make_async_copy(rows[2], acc_sh.at[dstv[2]], ssem[2]).wait()
    pltpu.make_async_copy(rows[0], acc_sh.at[dstv[0]], ssem[0]).wait()
    plsc.subcore_barrier()

    @pl.when(cid == 0)
    def _():
      pltpu.sync_copy(acc_sh.at[pl.ds(r0, _RPW)], out0.at[pl.ds(r0, _RPW)])

    @pl.when(cid == 1)
    def _():
      pltpu.sync_copy(acc_sh.at[pl.ds(r0, _RPW)], out1.at[pl.ds(r0, _RPW)])

  return seg_sum


_seg_sum_x = _make_seg_sum(_DF)
_seg_sum_b = _make_seg_sum(_FB)
_seg_sum_h = _make_seg_sum(_FH)


def _prep_body(bc_ref, ev_ref, out_ref):
  onehot = (bc_ref[...]
            == lax.broadcasted_iota(jnp.int32, (1, _G), 1)).astype(jnp.float32)
  rep = jnp.dot(onehot, ev_ref[...], preferred_element_type=jnp.float32)
  onescol = (lax.broadcasted_iota(jnp.int32, (_RT, 16), 1) == 0
             ).astype(jnp.float32)
  out_ref[...] = jnp.concatenate([rep, onescol], axis=1)


def _l0_body(p0x_ref, p1x_ref, p0b_ref, p1b_ref, tb_ref, x_ref,
             wlx_ref, wlr_ref, wrx_ref, wrr_ref, b_ref,
             h0a_ref, h0b_ref, aux_ref):
  sx = p0x_ref[...] + p1x_ref[...]
  sb = p0b_ref[...] + p1b_ref[...]
  dinv = 1.0 / jnp.maximum(sb[:, _DI:_DI + 1], 1.0)
  mrep = sb[:, 0:_DI] * dinv
  h0 = (jnp.dot(sx * dinv, wlx_ref[...], preferred_element_type=jnp.float32)
        + jnp.dot(mrep, wlr_ref[...], preferred_element_type=jnp.float32)
        + jnp.dot(x_ref[...], wrx_ref[...], preferred_element_type=jnp.float32)
        + jnp.dot(tb_ref[:, 0:_DI], wrr_ref[...],
                  preferred_element_type=jnp.float32)
        + b_ref[...])
  h0a_ref[...] = h0[:, :_FH]
  h0b_ref[...] = h0[:, _FH:]
  aux_ref[...] = jnp.concatenate(
      [mrep, jnp.broadcast_to(dinv, (_RT, 32))], axis=1)


def _l1_body(q0a_ref, q1a_ref, q0b_ref, q1b_ref, aux_ref, h0a_ref, h0b_ref,
             tb_ref, wlh_ref, wlr_ref, wrh_ref, wrr_ref, b_ref, out_ref):
  dinv = aux_ref[:, 32:33]
  mrep = aux_ref[:, 0:32]
  meanh = jnp.concatenate([(q0a_ref[...] + q1a_ref[...]) * dinv,
                           (q0b_ref[...] + q1b_ref[...]) * dinv], axis=1)
  h0 = jnp.concatenate([h0a_ref[...], h0b_ref[...]], axis=1)
  rep = tb_ref[:, 0:_DI]
  out_ref[...] = (
      jnp.dot(meanh, wlh_ref[...], preferred_element_type=jnp.float32)
      + jnp.dot(mrep, wlr_ref[...], preferred_element_type=jnp.float32)
      + jnp.dot(h0, wrh_ref[...], preferred_element_type=jnp.float32)
      + jnp.dot(rep, wrr_ref[...], preferred_element_type=jnp.float32)
      + b_ref[...])


def _pool_body(bc_ref, h0a_ref, h0b_ref, h1_ref,
               fc1w_ref, fc1b_ref, fc2w_ref, fc2b_ref, out_ref, pool_ref):
  i = pl.program_id(0)

  @pl.when(i == 0)
  def _():
    pool_ref[...] = jnp.full((_G, 2 * _OH), -jnp.inf, jnp.float32)

  hcat = jnp.concatenate([h0a_ref[...], h0b_ref[...], h1_ref[...]], axis=1)
  bc = bc_ref[...]

  def gbody(g, carry):
    red = jnp.max(jnp.where(bc == g, hcat, -jnp.inf), axis=0, keepdims=True)
    pool_ref[pl.ds(g, 1), :] = jnp.maximum(pool_ref[pl.ds(g, 1), :], red)
    return carry

  lax.fori_loop(0, _G, gbody, 0)

  @pl.when(i == pl.num_programs(0) - 1)
  def _():
    z = jnp.maximum(
        jnp.dot(pool_ref[...], fc1w_ref[...],
                preferred_element_type=jnp.float32) + fc1b_ref[...], 0.0)
    out_ref[...] = (jnp.dot(z, fc2w_ref[...],
                            preferred_element_type=jnp.float32)
                    + fc2b_ref[...])


def kernel(x, edge_index, batch, eigen_values,
           W_l0, b_l0, W_r0, W_l1, b_l1, W_r1,
           fc1_w, fc1_b, fc2_w, fc2_b):
  ng = _N // _RT
  srcp = jnp.concatenate(
      [edge_index[0], jnp.zeros((_EPAD - _E,), jnp.int32)])
  dstp = jnp.concatenate(
      [edge_index[1], jnp.full((_EPAD - _E,), _N, jnp.int32)])
  bc = batch.reshape(_N, 1)
  z_x = jnp.zeros((_NPAD, _DF), jnp.float32)
  z_b = jnp.zeros((_NPAD, _FB), jnp.float32)
  z_h = jnp.zeros((_NPAD, _FH), jnp.float32)
  b_l0r = b_l0.reshape(1, _OH)
  b_l1r = b_l1.reshape(1, _OH)
  fc1_br = fc1_b.reshape(1, -1)
  fc2_br = fc2_b.reshape(1, -1)
  wl0x, wl0r = W_l0[:_DF], W_l0[_DF:]
  wr0x, wr0r = W_r0[:_DF], W_r0[_DF:]
  wl1h, wl1r = W_l1[:_OH], W_l1[_OH:]
  wr1h, wr1r = W_r1[:_OH], W_r1[_OH:]

  tb = pl.pallas_call(
      _prep_body,
      grid=(ng,),
      in_specs=[
          pl.BlockSpec((_RT, 1), lambda i: (i, 0)),
          pl.BlockSpec((_G, _DI), lambda i: (0, 0)),
      ],
      out_specs=pl.BlockSpec((_RT, _FB), lambda i: (i, 0)),
      out_shape=jax.ShapeDtypeStruct((_N, _FB), jnp.float32),
  )(bc, eigen_values)

  p0x, p1x = _seg_sum_x(srcp, dstp, x, z_x)
  p0b, p1b = _seg_sum_b(srcp, dstp, tb, z_b)

  wspec = lambda r, c: pl.BlockSpec((r, c), lambda i: (0, 0))
  rspec = lambda c: pl.BlockSpec((_RT, c), lambda i: (i, 0))
  h0a, h0b, aux = pl.pallas_call(
      _l0_body,
      grid=(ng,),
      in_specs=[
          rspec(_DF), rspec(_DF), rspec(_FB), rspec(_FB), rspec(_FB),
          rspec(_DF),
          wspec(_DF, _OH), wspec(_DI, _OH),
          wspec(_DF, _OH), wspec(_DI, _OH), wspec(1, _OH),
      ],
      out_specs=[rspec(_FH), rspec(_FH), rspec(64)],
      out_shape=[
          jax.ShapeDtypeStruct((_N, _FH), jnp.float32),
          jax.ShapeDtypeStruct((_N, _FH), jnp.float32),
          jax.ShapeDtypeStruct((_N, 64), jnp.float32),
      ],
  )(p0x, p1x, p0b, p1b, tb, x, wl0x, wl0r, wr0x, wr0r, b_l0r)

  q0a, q1a = _seg_sum_h(srcp, dstp, h0a, z_h)
  q0b, q1b = _seg_sum_h(srcp, dstp, h0b, z_h)

  h1 = pl.pallas_call(
      _l1_body,
      grid=(ng,),
      in_specs=[
          rspec(_FH), rspec(_FH), rspec(_FH), rspec(_FH),
          rspec(64), rspec(_FH), rspec(_FH), rspec(_FB),
          wspec(_OH, _OH), wspec(_DI, _OH),
          wspec(_OH, _OH), wspec(_DI, _OH), wspec(1, _OH),
      ],
      out_specs=rspec(_OH),
      out_shape=jax.ShapeDtypeStruct((_N, _OH), jnp.float32),
  )(q0a, q1a, q0b, q1b, aux, h0a, h0b, tb,
    wl1h, wl1r, wr1h, wr1r, b_l1r)

  out = pl.pallas_call(
      _pool_body,
      grid=(ng,),
      in_specs=[
          pl.BlockSpec((_RT, 1), lambda i: (i, 0)),
          rspec(_FH), rspec(_FH), rspec(_OH),
          wspec(2 * _OH, 256), wspec(1, 256),
          wspec(256, 10), wspec(1, 10),
      ],
      out_specs=pl.BlockSpec((_G, 10), lambda i: (0, 0)),
      out_shape=jax.ShapeDtypeStruct((_G, 10), jnp.float32),
      scratch_shapes=[pltpu.VMEM((_G, 2 * _OH), jnp.float32)],
  )(bc, h0a, h0b, h1, fc1_w, fc1_br, fc2_w, fc2_br)
  return out


# speedup vs baseline: 4.8896x; 1.0983x over previous
import functools

import jax
import jax.numpy as jnp
from jax import lax
from jax.experimental import pallas as pl
from jax.experimental.pallas import tpu as pltpu
from jax.experimental.pallas import tpu_sc as plsc

_N = 10000
_E = 320000
_G = 64
_DF = 128
_DI = 32
_OH = 224
_FB = _DI + 16
_FH = _OH // 2

_NC = 2
_NS = 16
_NW = _NC * _NS
_CH = 128
_EPW = 10112
_EPAD = _NW * _EPW
_NCH = _EPW // _CH
_NCH0 = 112
_NCH1 = 46
_NPAD = 10112
_RPW = _NPAD // _NS

_RT = 2000


def _make_seg_sum(F):
  mesh = plsc.VectorSubcoreMesh(
      core_axis_name="c", subcore_axis_name="s",
      num_cores=_NC, num_subcores=_NS)

  @functools.partial(
      pl.kernel,
      out_type=(jax.ShapeDtypeStruct((_NPAD, F), jnp.float32),
                jax.ShapeDtypeStruct((_NPAD, F), jnp.float32)),
      mesh=mesh,
      compiler_params=pltpu.CompilerParams(use_tc_tiling_on_sc=False),
      scratch_types=[
          pltpu.VMEM((_CH,), jnp.int32),
          pltpu.VMEM((_CH,), jnp.int32),
          pltpu.VMEM((_CH,), jnp.int32),
          pltpu.VMEM((_CH,), jnp.int32),
          pltpu.VMEM((_CH,), jnp.int32),
          pltpu.VMEM((_CH,), jnp.int32),
          pltpu.VMEM((_CH, F), jnp.float32),
          pltpu.VMEM((_CH, F), jnp.float32),
          pltpu.VMEM((_CH, F), jnp.float32),
          pltpu.SemaphoreType.DMA,
          pltpu.SemaphoreType.DMA,
          pltpu.SemaphoreType.DMA,
          pltpu.SemaphoreType.DMA,
          pltpu.SemaphoreType.DMA,
          pltpu.SemaphoreType.DMA,
          pltpu.SemaphoreType.DMA,
          pltpu.SemaphoreType.DMA,
          pltpu.SemaphoreType.DMA,
          pltpu.VMEM_SHARED((_NPAD, F), jnp.float32),
      ])
  def seg_sum(src_hbm, dst_hbm, table_hbm, zeros_hbm, out0, out1,
              src_0, src_1, src_2, dst_0, dst_1, dst_2,
              rows_0, rows_1, rows_2,
              gsem_0, gsem_1, gsem_2, ssem_0, ssem_1, ssem_2,
              isem_0, isem_1, isem_2, acc_sh):
    srcv = (src_0, src_1, src_2)
    dstv = (dst_0, dst_1, dst_2)
    rows = (rows_0, rows_1, rows_2)
    gsem = (gsem_0, gsem_1, gsem_2)
    ssem = (ssem_0, ssem_1, ssem_2)
    isem = (isem_0, isem_1, isem_2)
    cid = lax.axis_index("c")
    sid = lax.axis_index("s")
    nch_w = jnp.where(cid == 0, _NCH0, _NCH1)
    c_base = jnp.where(cid == 0, sid * _NCH0, _NS * _NCH0 + sid * _NCH1)
    e0 = c_base * _CH

    def fetch(c, k):
      pltpu.async_copy(src_hbm.at[pl.ds(e0 + c * _CH, _CH)], srcv[k], isem[k])
      pltpu.async_copy(dst_hbm.at[pl.ds(e0 + c * _CH, _CH)], dstv[k], isem[k])

    def fetch_wait(k):
      pltpu.make_async_copy(src_hbm.at[0:_CH], srcv[k], isem[k]).wait()
      pltpu.make_async_copy(src_hbm.at[0:_CH], dstv[k], isem[k]).wait()

    fetch(0, 0)
    r0 = sid * _RPW
    pltpu.sync_copy(zeros_hbm.at[pl.ds(r0, _RPW)], acc_sh.at[pl.ds(r0, _RPW)])
    plsc.subcore_barrier()

    def step(c, k, kp, kn, g2, g1):
      fetch_wait(k)
      pltpu.async_copy(table_hbm.at[srcv[k]], rows[k], gsem[k])

      @pl.when(g2)
      def _():
        pltpu.make_async_copy(rows[kn], acc_sh.at[dstv[kn]], ssem[kn]).wait()
      fetch(c + 1, kn)

      @pl.when(g1)
      def _():
        pltpu.make_async_copy(table_hbm.at[0:_CH], rows[kp], gsem[kp]).wait()
        pltpu.async_copy(rows[kp], acc_sh.at[dstv[kp]], ssem[kp], add=True)

    def body(j, carry):
      nf = j > 0
      step(3 * j, 0, 2, 1, nf, nf)
      step(3 * j + 1, 1, 0, 2, nf, jnp.bool_(True))
      step(3 * j + 2, 2, 1, 0, jnp.bool_(True), jnp.bool_(True))
      return carry

    lax.fori_loop(0, (nch_w - 1) // 3, body, 0)
    fetch_wait(0)
    pltpu.async_copy(table_hbm.at[srcv[0]], rows[0], gsem[0])
    pltpu.make_async_copy(rows[1], acc_sh.at[dstv[1]], ssem[1]).wait()
    pltpu.make_async_copy(table_hbm.at[0:_CH], rows[2], gsem[2]).wait()
    pltpu.async_copy(rows[2], acc_sh.at[dstv[2]], ssem[2], add=True)
    pltpu.make_async_copy(table_hbm.at[0:_CH], rows[0], gsem[0]).wait()
    pltpu.async_copy(rows[0], acc_sh.at[dstv[0]], ssem[0], add=True)
    pltpu.make_async_copy(rows[2], acc_sh.at[dstv[2]], ssem[2]).wait()
    pltpu.make_async_copy(rows[0], acc_sh.at[dstv[0]], ssem[0]).wait()
    plsc.subcore_barrier()

    @pl.when(cid == 0)
    def _():
      pltpu.sync_copy(acc_sh.at[pl.ds(r0, _RPW)], out0.at[pl.ds(r0, _RPW)])

    @pl.when(cid == 1)
    def _():
      pltpu.sync_copy(acc_sh.at[pl.ds(r0, _RPW)], out1.at[pl.ds(r0, _RPW)])

  return seg_sum


_seg_sum_x = _make_seg_sum(_DF)
_seg_sum_b = _make_seg_sum(_FB)
_seg_sum_h = _make_seg_sum(_FH)


def _prep_body(bc_ref, ev_ref, out_ref):
  onehot = (bc_ref[...]
            == lax.broadcasted_iota(jnp.int32, (1, _G), 1)).astype(jnp.float32)
  rep = jnp.dot(onehot, ev_ref[...], preferred_element_type=jnp.float32)
  onescol = (lax.broadcasted_iota(jnp.int32, (_RT, 16), 1) == 0
             ).astype(jnp.float32)
  out_ref[...] = jnp.concatenate([rep, onescol], axis=1)


def _l0_body(p0x_ref, p1x_ref, p0b_ref, p1b_ref, tb_ref, x_ref,
             wlx_ref, wlr_ref, wrx_ref, wrr_ref, b_ref,
             h0a_ref, h0b_ref, aux_ref):
  sx = p0x_ref[...] + p1x_ref[...]
  sb = p0b_ref[...] + p1b_ref[...]
  dinv = 1.0 / jnp.maximum(sb[:, _DI:_DI + 1], 1.0)
  mrep = sb[:, 0:_DI] * dinv
  h0 = (jnp.dot(sx * dinv, wlx_ref[...], preferred_element_type=jnp.float32)
        + jnp.dot(mrep, wlr_ref[...], preferred_element_type=jnp.float32)
        + jnp.dot(x_ref[...], wrx_ref[...], preferred_element_type=jnp.float32)
        + jnp.dot(tb_ref[:, 0:_DI], wrr_ref[...],
                  preferred_element_type=jnp.float32)
        + b_ref[...])
  h0a_ref[...] = h0[:, :_FH]
  h0b_ref[...] = h0[:, _FH:]
  aux_ref[...] = jnp.concatenate(
      [mrep, jnp.broadcast_to(dinv, (_RT, 32))], axis=1)


def _l1_body(q0a_ref, q1a_ref, q0b_ref, q1b_ref, aux_ref, h0a_ref, h0b_ref,
             tb_ref, wlh_ref, wlr_ref, wrh_ref, wrr_ref, b_ref, out_ref):
  dinv = aux_ref[:, 32:33]
  mrep = aux_ref[:, 0:32]
  meanh = jnp.concatenate([(q0a_ref[...] + q1a_ref[...]) * dinv,
                           (q0b_ref[...] + q1b_ref[...]) * dinv], axis=1)
  h0 = jnp.concatenate([h0a_ref[...], h0b_ref[...]], axis=1)
  rep = tb_ref[:, 0:_DI]
  out_ref[...] = (
      jnp.dot(meanh, wlh_ref[...], preferred_element_type=jnp.float32)
      + jnp.dot(mrep, wlr_ref[...], preferred_element_type=jnp.float32)
      + jnp.dot(h0, wrh_ref[...], preferred_element_type=jnp.float32)
      + jnp.dot(rep, wrr_ref[...], preferred_element_type=jnp.float32)
      + b_ref[...])


def _pool_body(bc_ref, h0a_ref, h0b_ref, h1_ref,
               fc1w_ref, fc1b_ref, fc2w_ref, fc2b_ref, out_ref, pool_ref):
  i = pl.program_id(0)

  @pl.when(i == 0)
  def _():
    pool_ref[...] = jnp.full((_G, 2 * _OH), -jnp.inf, jnp.float32)

  hcat = jnp.concatenate([h0a_ref[...], h0b_ref[...], h1_ref[...]], axis=1)
  bc = bc_ref[...]

  def gbody(g, carry):
    red = jnp.max(jnp.where(bc == g, hcat, -jnp.inf), axis=0, keepdims=True)
    pool_ref[pl.ds(g, 1), :] = jnp.maximum(pool_ref[pl.ds(g, 1), :], red)
    return carry

  lax.fori_loop(0, _G, gbody, 0)

  @pl.when(i == pl.num_programs(0) - 1)
  def _():
    z = jnp.maximum(
        jnp.dot(pool_ref[...], fc1w_ref[...],
                preferred_element_type=jnp.float32) + fc1b_ref[...], 0.0)
    out_ref[...] = (jnp.dot(z, fc2w_ref[...],
                            preferred_element_type=jnp.float32)
                    + fc2b_ref[...])


def kernel(x, edge_index, batch, eigen_values,
           W_l0, b_l0, W_r0, W_l1, b_l1, W_r1,
           fc1_w, fc1_b, fc2_w, fc2_b):
  ng = _N // _RT
  srcp = jnp.concatenate(
      [edge_index[0], jnp.zeros((_EPAD - _E,), jnp.int32)])
  dstp = jnp.concatenate(
      [edge_index[1], jnp.full((_EPAD - _E,), _N, jnp.int32)])
  bc = batch.reshape(_N, 1)
  z_x = jnp.zeros((_NPAD, _DF), jnp.float32)
  z_b = jnp.zeros((_NPAD, _FB), jnp.float32)
  z_h = jnp.zeros((_NPAD, _FH), jnp.float32)
  b_l0r = b_l0.reshape(1, _OH)
  b_l1r = b_l1.reshape(1, _OH)
  fc1_br = fc1_b.reshape(1, -1)
  fc2_br = fc2_b.reshape(1, -1)
  wl0x, wl0r = W_l0[:_DF], W_l0[_DF:]
  wr0x, wr0r = W_r0[:_DF], W_r0[_DF:]
  wl1h, wl1r = W_l1[:_OH], W_l1[_OH:]
  wr1h, wr1r = W_r1[:_OH], W_r1[_OH:]

  tb = pl.pallas_call(
      _prep_body,
      grid=(ng,),
      in_specs=[
          pl.BlockSpec((_RT, 1), lambda i: (i, 0)),
          pl.BlockSpec((_G, _DI), lambda i: (0, 0)),
      ],
      out_specs=pl.BlockSpec((_RT, _FB), lambda i: (i, 0)),
      out_shape=jax.ShapeDtypeStruct((_N, _FB), jnp.float32),
  )(bc, eigen_values)

  p0x, p1x = _seg_sum_x(srcp, dstp, x, z_x)
  p0b, p1b = _seg_sum_b(srcp, dstp, tb, z_b)

  wspec = lambda r, c: pl.BlockSpec((r, c), lambda i: (0, 0))
  rspec = lambda c: pl.BlockSpec((_RT, c), lambda i: (i, 0))
  h0a, h0b, aux = pl.pallas_call(
      _l0_body,
      grid=(ng,),
      in_specs=[
          rspec(_DF), rspec(_DF), rspec(_FB), rspec(_FB), rspec(_FB),
          rspec(_DF),
          wspec(_DF, _OH), wspec(_DI, _OH),
          wspec(_DF, _OH), wspec(_DI, _OH), wspec(1, _OH),
      ],
      out_specs=[rspec(_FH), rspec(_FH), rspec(64)],
      out_shape=[
          jax.ShapeDtypeStruct((_N, _FH), jnp.float32),
          jax.ShapeDtypeStruct((_N, _FH), jnp.float32),
          jax.ShapeDtypeStruct((_N, 64), jnp.float32),
      ],
  )(p0x, p1x, p0b, p1b, tb, x, wl0x, wl0r, wr0x, wr0r, b_l0r)

  q0a, q1a = _seg_sum_h(srcp, dstp, h0a, z_h)
  q0b, q1b = _seg_sum_h(srcp, dstp, h0b, z_h)

  h1 = pl.pallas_call(
      _l1_body,
      grid=(ng,),
      in_specs=[
          rspec(_FH), rspec(_FH), rspec(_FH), rspec(_FH),
          rspec(64), rspec(_FH), rspec(_FH), rspec(_FB),
          wspec(_OH, _OH), wspec(_DI, _OH),
          wspec(_OH, _OH), wspec(_DI, _OH), wspec(1, _OH),
      ],
      out_specs=rspec(_OH),
      out_shape=jax.ShapeDtypeStruct((_N, _OH), jnp.float32),
  )(q0a, q1a, q0b, q1b, aux, h0a, h0b, tb,
    wl1h, wl1r, wr1h, wr1r, b_l1r)

  out = pl.pallas_call(
      _pool_body,
      grid=(ng,),
      in_specs=[
          pl.BlockSpec((_RT, 1), lambda i: (i, 0)),
          rspec(_FH), rspec(_FH), rspec(_OH),
          wspec(2 * _OH, 256), wspec(1, 256),
          wspec(256, 10), wspec(1, 10),
      ],
      out_specs=pl.BlockSpec((_G, 10), lambda i: (0, 0)),
      out_shape=jax.ShapeDtypeStruct((_G, 10), jnp.float32),
      scratch_shapes=[pltpu.VMEM((_G, 2 * _OH), jnp.float32)],
  )(bc, h0a, h0b, h1, fc1_w, fc1_br, fc2_w, fc2_br)
  return out
